# baseline (speedup 1.0000x reference)
import functools
import jax
import jax.numpy as jnp
from jax import lax
from jax.experimental import pallas as pl
from jax.experimental.pallas import tpu as pltpu
from jax.experimental.pallas import tpu_sc as plsc

EPS = 1e-6
HW = 128
OUT = HW * HW
L = 16
NC, NS = 2, 16
WPB = 8
NROW = 32

_SC_PARAMS = pltpu.CompilerParams(
    needs_layout_passes=False, use_tc_tiling_on_sc=False)
_MESH = plsc.VectorSubcoreMesh(core_axis_name="c", subcore_axis_name="s")


def _tc_weights_body(idx_ref, mt_ref, st_ref, v_ref, w_ref):
    pts = idx_ref[0].astype(jnp.float32)
    rank = pts.shape[1]
    acc = None
    for r in range(rank):
        p = pts[:, r:r + 1]
        m = mt_ref[0, r:r + 1, :]
        sg = st_ref[0, r:r + 1, :]
        d = p - m
        t = d * d * (1.0 / (EPS + sg))
        acc = t if acc is None else acc + t
    props = jnp.exp(-0.5 * acc)
    denom = jnp.sum(props, axis=0, keepdims=True) + EPS
    vp = v_ref[0] / denom
    wcol = jnp.sum(props * vp, axis=1, keepdims=True)
    w_ref[0] = wcol.reshape(NROW, HW)


def _tc_weights(indices, means_t, sig_t, vals3):
    b, n, rank = indices.shape
    k = means_t.shape[2]
    return pl.pallas_call(
        _tc_weights_body,
        grid=(b,),
        in_specs=[
            pl.BlockSpec((1, n, rank), lambda i: (i, 0, 0)),
            pl.BlockSpec((1, rank, k), lambda i: (i, 0, 0)),
            pl.BlockSpec((1, rank, k), lambda i: (i, 0, 0)),
            pl.BlockSpec((1, 1, k), lambda i: (i, 0, 0)),
        ],
        out_specs=pl.BlockSpec((1, NROW, HW), lambda i: (i, 0, 0)),
        out_shape=jax.ShapeDtypeStruct((b, NROW, HW), jnp.float32),
    )(indices, means_t, sig_t, vals3)


def _sc_gather(xflat, indices):
    b, n, rank = indices.shape
    ppw = n // WPB
    rpw4 = NROW // WPB
    bpc = b // NC
    groups = ppw // L

    @functools.partial(
        pl.kernel,
        out_type=[
            jax.ShapeDtypeStruct((b, NROW, HW), jnp.int32),
            jax.ShapeDtypeStruct((b, NROW, HW), jnp.float32),
        ],
        mesh=_MESH,
        compiler_params=_SC_PARAMS,
        scratch_types=[
            pltpu.VMEM((OUT,), jnp.float32),
            pltpu.VMEM((ppw, 4), jnp.int32),
            pltpu.VMEM((rpw4, HW), jnp.int32),
            pltpu.VMEM((rpw4, HW), jnp.float32),
            pltpu.SemaphoreType.DMA,
        ],
    )
    def k(xf, idx, o3, g3, x_v, if_v, o_out_v, g_out_v, sem):
        c = lax.axis_index("c")
        s = lax.axis_index("s")
        b_local = s // WPB
        bat = c * bpc + b_local
        chunk = s % WPB
        pbase = chunk * ppw

        cp_x = pltpu.async_copy(xf.at[bat], x_v, sem)
        cp_i = pltpu.async_copy(
            idx.at[bat, pl.ds(pbase, ppw), :], if_v, sem)
        cp_x.wait()
        cp_i.wait()

        iot = lax.iota(jnp.int32, L)
        c0 = jnp.zeros((L,), jnp.int32)
        for r in range(rpw4):
            for l in range(HW // L):
                pt = (r * (HW // L) + l) * L + iot
                i0 = plsc.load_gather(if_v, [pt, c0])
                i1 = plsc.load_gather(if_v, [pt, c0 + 1])
                i2 = plsc.load_gather(if_v, [pt, c0 + 2])
                i3 = plsc.load_gather(if_v, [pt, c0 + 3])
                sl = pl.ds(l * L, L)
                o_out_v[r, sl] = i0 * HW + i1
                g_out_v[r, sl] = plsc.load_gather(x_v, [i2 * HW + i3])
        rbase4 = chunk * rpw4
        pltpu.sync_copy(o_out_v, o3.at[bat, pl.ds(rbase4, rpw4), :])
        pltpu.sync_copy(g_out_v, g3.at[bat, pl.ds(rbase4, rpw4), :])

    return k(xflat, indices)


def _sc_scatter(w3, o3, g3, rowidx):
    b = w3.shape[0]
    rpw4 = NROW // WPB
    rows = OUT // L
    rpw = rows // WPB
    bpc = b // NC

    @functools.partial(
        pl.kernel,
        out_type=jax.ShapeDtypeStruct((b, rows, L), jnp.float32),
        mesh=_MESH,
        compiler_params=_SC_PARAMS,
        scratch_types=[
            pltpu.VMEM((rows, L), jnp.float32),
            pltpu.VMEM((rpw4, HW), jnp.float32),
            pltpu.VMEM((rpw4, HW), jnp.int32),
            pltpu.VMEM((rpw4, HW), jnp.float32),
            pltpu.VMEM((WPB, HW), jnp.int32),
            pltpu.VMEM_SHARED((bpc * rows, L), jnp.float32),
            pltpu.SemaphoreType.DMA,
        ],
    )
    def k(w_all, o_all, g_all, ridx, out,
          y_v, w_v, o_v, g_v, rowi_v, shared, sem):
        c = lax.axis_index("c")
        s = lax.axis_index("s")
        b_local = s // WPB
        bat = c * bpc + b_local
        chunk = s % WPB
        rbase4 = chunk * rpw4

        cp_w = pltpu.async_copy(
            w_all.at[bat, pl.ds(rbase4, rpw4), :], w_v, sem)
        cp_o = pltpu.async_copy(
            o_all.at[bat, pl.ds(rbase4, rpw4), :], o_v, sem)
        cp_g = pltpu.async_copy(
            g_all.at[bat, pl.ds(rbase4, rpw4), :], g_v, sem)
        cp_r = pltpu.async_copy(ridx.at[b_local], rowi_v, sem)

        zero = jnp.zeros((L,), jnp.float32)

        def zr(i, _):
            base = i * 8
            for u in range(8):
                y_v[base + u, :] = zero
            return 0
        lax.fori_loop(0, rows // 8, zr, 0)

        cp_w.wait()
        cp_o.wait()
        cp_g.wait()
        cp_r.wait()

        for r in range(rpw4):
            for l in range(HW // L):
                sl = pl.ds(l * L, L)
                o = o_v[r, sl]
                val = g_v[r, sl] * w_v[r, sl]
                plsc.addupdate_scatter(y_v, [o >> 4, o & 15], val)

        @pl.when(chunk == 0)
        def _():
            pltpu.sync_copy(y_v, shared.at[pl.ds(b_local * rows, rows)])
        plsc.subcore_barrier()

        @pl.when(chunk > 0)
        def _():
            for j in range(WPB):
                pltpu.sync_copy(y_v.at[pl.ds(j * HW, HW)],
                                shared.at[rowi_v.at[j]], add=True)
        plsc.subcore_barrier()

        rbase = chunk * rpw
        pltpu.sync_copy(shared.at[pl.ds(b_local * rows + rbase, rpw)],
                        out.at[bat, pl.ds(rbase, rpw)])

    return k(w3, o3, g3, rowidx)


def kernel(x, means, sigmas, values, indices):
    b, h, w = x.shape
    k = means.shape[1]
    xflat = x.reshape(b, h * w)
    means_t = means.transpose(0, 2, 1)
    sig_t = sigmas.transpose(0, 2, 1)
    vals3 = values.reshape(b, 1, k)

    w3 = _tc_weights(indices, means_t, sig_t, vals3)
    o3, g3 = _sc_gather(xflat, indices)

    rows = (h * w) // L
    bpc = b // NC
    rowidx = (jnp.arange(bpc, dtype=jnp.int32)[:, None, None] * rows
              + jnp.arange(rows, dtype=jnp.int32).reshape(WPB, HW)[None])
    y = _sc_scatter(w3, o3, g3, rowidx)
    return y.reshape(b, h, w)


# speedup vs baseline: 1.0791x; 1.0791x over previous
import functools
import jax
import jax.numpy as jnp
from jax import lax
from jax.experimental import pallas as pl
from jax.experimental.pallas import tpu as pltpu
from jax.experimental.pallas import tpu_sc as plsc

EPS = 1e-6
HW = 128
OUT = HW * HW
L = 16
NC, NS = 2, 16
WPB = 8
NROW = 32


def _tc_weights_body(idx_ref, mt_ref, st_ref, v_ref, w_ref, og_ref):
    idx = idx_ref[0]
    pts = idx.astype(jnp.float32)
    rank = pts.shape[1]
    acc = None
    for r in range(rank):
        p = pts[:, r:r + 1]
        m = mt_ref[0, r:r + 1, :]
        sg = st_ref[0, r:r + 1, :]
        d = p - m
        t = d * d * (1.0 / (EPS + sg))
        acc = t if acc is None else acc + t
    props = jnp.exp(-0.5 * acc)
    denom = jnp.sum(props, axis=0, keepdims=True) + EPS
    vp = v_ref[0] / denom
    wcol = jnp.sum(props * vp, axis=1, keepdims=True)
    w_ref[0] = wcol.reshape(NROW, HW)
    ogcol = ((idx[:, 0:1] * HW + idx[:, 1:2]) * OUT
             + idx[:, 2:3] * HW + idx[:, 3:4])
    og_ref[0] = ogcol.reshape(NROW, HW)


def _tc_weights(indices, means_t, sig_t, vals3):
    b, n, rank = indices.shape
    k = means_t.shape[2]
    return pl.pallas_call(
        _tc_weights_body,
        grid=(b,),
        in_specs=[
            pl.BlockSpec((1, n, rank), lambda i: (i, 0, 0)),
            pl.BlockSpec((1, rank, k), lambda i: (i, 0, 0)),
            pl.BlockSpec((1, rank, k), lambda i: (i, 0, 0)),
            pl.BlockSpec((1, 1, k), lambda i: (i, 0, 0)),
        ],
        out_specs=[
            pl.BlockSpec((1, NROW, HW), lambda i: (i, 0, 0)),
            pl.BlockSpec((1, NROW, HW), lambda i: (i, 0, 0)),
        ],
        out_shape=[
            jax.ShapeDtypeStruct((b, NROW, HW), jnp.float32),
            jax.ShapeDtypeStruct((b, NROW, HW), jnp.int32),
        ],
    )(indices, means_t, sig_t, vals3)


def _sc_scatter(xflat, w3, og3, rowidx):
    b = xflat.shape[0]
    rpw4 = NROW // WPB
    rows = OUT // L
    rpw = rows // WPB
    bpc = b // NC
    mesh = plsc.VectorSubcoreMesh(core_axis_name="c", subcore_axis_name="s")

    @functools.partial(
        pl.kernel,
        out_type=jax.ShapeDtypeStruct((b, rows, L), jnp.float32),
        mesh=mesh,
        compiler_params=pltpu.CompilerParams(
            needs_layout_passes=False, use_tc_tiling_on_sc=False),
        scratch_types=[
            pltpu.VMEM((OUT,), jnp.float32),
            pltpu.VMEM((rows, L), jnp.float32),
            pltpu.VMEM((rpw4, HW), jnp.float32),
            pltpu.VMEM((rpw4, HW), jnp.int32),
            pltpu.VMEM((WPB, HW), jnp.int32),
            pltpu.VMEM_SHARED((bpc * rows, L), jnp.float32),
            pltpu.SemaphoreType.DMA,
        ],
    )
    def sc_kernel(xf, w_all, og_all, ridx, out,
                  x_v, y_v, w_v, og_v, rowi_v, shared, sem):
        c = lax.axis_index("c")
        s = lax.axis_index("s")
        b_local = s // WPB
        bat = c * bpc + b_local
        chunk = s % WPB
        rbase4 = chunk * rpw4

        cp_x = pltpu.async_copy(xf.at[bat], x_v, sem)
        cp_w = pltpu.async_copy(
            w_all.at[bat, pl.ds(rbase4, rpw4), :], w_v, sem)
        cp_o = pltpu.async_copy(
            og_all.at[bat, pl.ds(rbase4, rpw4), :], og_v, sem)
        cp_r = pltpu.async_copy(ridx.at[b_local], rowi_v, sem)

        zero = jnp.zeros((L,), jnp.float32)

        def zr(i, _):
            base = i * 8
            for u in range(8):
                y_v[base + u, :] = zero
            return 0
        lax.fori_loop(0, rows // 8, zr, 0)

        cp_x.wait()
        cp_w.wait()
        cp_o.wait()
        cp_r.wait()

        for r in range(rpw4):
            for l in range(HW // L):
                sl = pl.ds(l * L, L)
                og = og_v[r, sl]
                o = og >> 14
                gi = og & (OUT - 1)
                gx = plsc.load_gather(x_v, [gi])
                val = gx * w_v[r, sl]
                plsc.addupdate_scatter(y_v, [o >> 4, o & 15], val)

        @pl.when(chunk == 0)
        def _():
            pltpu.sync_copy(y_v, shared.at[pl.ds(b_local * rows, rows)])
        plsc.subcore_barrier()

        @pl.when(chunk > 0)
        def _():
            for j in range(WPB):
                pltpu.sync_copy(y_v.at[pl.ds(j * HW, HW)],
                                shared.at[rowi_v.at[j]], add=True)
        plsc.subcore_barrier()

        rbase = chunk * rpw
        pltpu.sync_copy(shared.at[pl.ds(b_local * rows + rbase, rpw)],
                        out.at[bat, pl.ds(rbase, rpw)])

    return sc_kernel(xflat, w3, og3, rowidx)


def kernel(x, means, sigmas, values, indices):
    b, h, w = x.shape
    k = means.shape[1]
    xflat = x.reshape(b, h * w)
    means_t = means.transpose(0, 2, 1)
    sig_t = sigmas.transpose(0, 2, 1)
    vals3 = values.reshape(b, 1, k)

    w3, og3 = _tc_weights(indices, means_t, sig_t, vals3)

    rows = (h * w) // L
    bpc = b // NC
    rowidx = (jnp.arange(bpc, dtype=jnp.int32)[:, None, None] * rows
              + jnp.arange(rows, dtype=jnp.int32).reshape(WPB, HW)[None])
    y = _sc_scatter(xflat, w3, og3, rowidx)
    return y.reshape(b, h, w)


# speedup vs baseline: 1.1255x; 1.0430x over previous
import functools
import jax
import jax.numpy as jnp
from jax import lax
from jax.experimental import pallas as pl
from jax.experimental.pallas import tpu as pltpu
from jax.experimental.pallas import tpu_sc as plsc

EPS = 1e-6
HW = 128
OUT = HW * HW
L = 16
NC, NS = 2, 16
WPB = 8
NROW = 32


def _tc_weights_body(idx_ref, mt_ref, st_ref, v_ref, w_ref, o_ref, g_ref):
    idx = idx_ref[0]
    pts = idx.astype(jnp.float32)
    rank = pts.shape[1]
    acc = None
    for r in range(rank):
        p = pts[:, r:r + 1]
        m = mt_ref[0, r:r + 1, :]
        sg = st_ref[0, r:r + 1, :]
        d = p - m
        t = d * d * (1.0 / (EPS + sg))
        acc = t if acc is None else acc + t
    props = jnp.exp(-0.5 * acc)
    denom = jnp.sum(props, axis=0, keepdims=True) + EPS
    vp = v_ref[0] / denom
    wcol = jnp.sum(props * vp, axis=1, keepdims=True)
    w_ref[0] = wcol.reshape(NROW, HW)
    ocol = idx[:, 0:1] * HW + idx[:, 1:2]
    gcol = idx[:, 2:3] * HW + idx[:, 3:4]
    o_ref[0] = ocol.reshape(NROW, HW)
    g_ref[0] = gcol.reshape(NROW, HW)


def _tc_weights(indices, means_t, sig_t, vals3):
    b, n, rank = indices.shape
    k = means_t.shape[2]
    return pl.pallas_call(
        _tc_weights_body,
        grid=(b,),
        in_specs=[
            pl.BlockSpec((1, n, rank), lambda i: (i, 0, 0)),
            pl.BlockSpec((1, rank, k), lambda i: (i, 0, 0)),
            pl.BlockSpec((1, rank, k), lambda i: (i, 0, 0)),
            pl.BlockSpec((1, 1, k), lambda i: (i, 0, 0)),
        ],
        out_specs=[
            pl.BlockSpec((1, NROW, HW), lambda i: (i, 0, 0)),
            pl.BlockSpec((1, NROW, HW), lambda i: (i, 0, 0)),
            pl.BlockSpec((1, NROW, HW), lambda i: (i, 0, 0)),
        ],
        out_shape=[
            jax.ShapeDtypeStruct((b, NROW, HW), jnp.float32),
            jax.ShapeDtypeStruct((b, NROW, HW), jnp.int32),
            jax.ShapeDtypeStruct((b, NROW, HW), jnp.int32),
        ],
    )(indices, means_t, sig_t, vals3)


def _sc_scatter(xflat, w3, o3, g3, rowidx):
    b = xflat.shape[0]
    rpw4 = NROW // WPB
    rows = OUT // L
    rpw = rows // WPB
    bpc = b // NC
    mesh = plsc.VectorSubcoreMesh(core_axis_name="c", subcore_axis_name="s")

    @functools.partial(
        pl.kernel,
        out_type=jax.ShapeDtypeStruct((b, rows, L), jnp.float32),
        mesh=mesh,
        compiler_params=pltpu.CompilerParams(
            needs_layout_passes=False, use_tc_tiling_on_sc=False),
        scratch_types=[
            pltpu.VMEM((OUT,), jnp.float32),
            pltpu.VMEM((rows, L), jnp.float32),
            pltpu.VMEM((rpw4, HW), jnp.float32),
            pltpu.VMEM((rpw4, HW), jnp.int32),
            pltpu.VMEM((rpw4, HW), jnp.int32),
            pltpu.VMEM((WPB, HW), jnp.int32),
            pltpu.VMEM_SHARED((bpc * rows, L), jnp.float32),
            pltpu.SemaphoreType.DMA,
        ],
    )
    def sc_kernel(xf, w_all, o_all, g_all, ridx, out,
                  x_v, y_v, w_v, o_v, g_v, rowi_v, shared, sem):
        c = lax.axis_index("c")
        s = lax.axis_index("s")
        b_local = s // WPB
        bat = c * bpc + b_local
        chunk = s % WPB
        rbase4 = chunk * rpw4

        cp_x = pltpu.async_copy(xf.at[bat], x_v, sem)
        cp_w = pltpu.async_copy(
            w_all.at[bat, pl.ds(rbase4, rpw4), :], w_v, sem)
        cp_o = pltpu.async_copy(
            o_all.at[bat, pl.ds(rbase4, rpw4), :], o_v, sem)
        cp_g = pltpu.async_copy(
            g_all.at[bat, pl.ds(rbase4, rpw4), :], g_v, sem)
        cp_r = pltpu.async_copy(ridx.at[b_local], rowi_v, sem)

        zero = jnp.zeros((L,), jnp.float32)

        def zr(i, _):
            base = i * 8
            for u in range(8):
                y_v[base + u, :] = zero
            return 0
        lax.fori_loop(0, rows // 8, zr, 0)

        cp_x.wait()
        cp_w.wait()
        cp_o.wait()
        cp_g.wait()
        cp_r.wait()

        for r in range(rpw4):
            for l in range(HW // L):
                sl = pl.ds(l * L, L)
                o = o_v[r, sl]
                gi = g_v[r, sl]
                gx = plsc.load_gather(x_v, [gi])
                val = gx * w_v[r, sl]
                plsc.addupdate_scatter(y_v, [o >> 4, o & 15], val)

        @pl.when(chunk == 0)
        def _():
            pltpu.sync_copy(y_v, shared.at[pl.ds(b_local * rows, rows)])
        plsc.subcore_barrier()

        @pl.when(chunk > 0)
        def _():
            for j in range(WPB):
                pltpu.sync_copy(y_v.at[pl.ds(j * HW, HW)],
                                shared.at[rowi_v.at[j]], add=True)
        plsc.subcore_barrier()

        rbase = chunk * rpw
        pltpu.sync_copy(shared.at[pl.ds(b_local * rows + rbase, rpw)],
                        out.at[bat, pl.ds(rbase, rpw)])

    return sc_kernel(xflat, w3, o3, g3, rowidx)


def kernel(x, means, sigmas, values, indices):
    b, h, w = x.shape
    k = means.shape[1]
    xflat = x.reshape(b, h * w)
    means_t = means.transpose(0, 2, 1)
    sig_t = sigmas.transpose(0, 2, 1)
    vals3 = values.reshape(b, 1, k)

    w3, o3, g3 = _tc_weights(indices, means_t, sig_t, vals3)

    rows = (h * w) // L
    bpc = b // NC
    rowidx = (jnp.arange(bpc, dtype=jnp.int32)[:, None, None] * rows
              + jnp.arange(rows, dtype=jnp.int32).reshape(WPB, HW)[None])
    y = _sc_scatter(xflat, w3, o3, g3, rowidx)
    return y.reshape(b, h, w)
